# baseline (speedup 1.0000x reference)
import functools
import math

import jax
import jax.numpy as jnp
from jax import lax
from jax.experimental import pallas as pl
from jax.experimental.pallas import tpu as pltpu
import jax.experimental.pallas.tpu_sc as plsc

N_IN = 1000000
NPAD = 1 << 20
NROWS = NPAD // 128
BETA = 2.0 / 3.0
BIAS = -BETA * math.log(0.1 / 1.1)

NW = 32
PER_W = NPAD // NW
CHUNK = 2048
NCHUNK = PER_W // CHUNK
NB = 1024
HL = NB * 16

BLK0 = 512
GRID0 = NROWS // BLK0


def _sigmoid(z):
    return 1.0 / (1.0 + jnp.exp(-z))


def _k0_body(x_ref, bits_ref, l0_ref, acc_ref):
    pid = pl.program_id(0)

    @pl.when(pid == 0)
    def _():
        acc_ref[0] = 0.0

    x = x_ref[...]
    s1 = _sigmoid(x + BIAS)
    r = lax.broadcasted_iota(jnp.int32, (BLK0, 128), 0)
    c = lax.broadcasted_iota(jnp.int32, (BLK0, 128), 1)
    flat = (pid * BLK0 + r) * 128 + c
    s1 = jnp.where(flat < N_IN, s1, 0.0)
    acc_ref[0] += jnp.sum(s1)
    bits_ref[...] = lax.bitcast_convert_type(_sigmoid((x / BETA) * 0.8),
                                             jnp.int32)

    @pl.when(pid == pl.num_programs(0) - 1)
    def _():
        l0_ref[0] = acc_ref[0]


def _k0(x2d):
    return pl.pallas_call(
        _k0_body,
        grid=(GRID0,),
        in_specs=[pl.BlockSpec((BLK0, 128), lambda i: (i, 0))],
        out_specs=[
            pl.BlockSpec((BLK0, 128), lambda i: (i, 0)),
            pl.BlockSpec(memory_space=pltpu.SMEM),
        ],
        out_shape=[
            jax.ShapeDtypeStruct((NROWS, 128), jnp.int32),
            jax.ShapeDtypeStruct((1,), jnp.float32),
        ],
        scratch_shapes=[pltpu.SMEM((1,), jnp.float32)],
    )(x2d)


_SC_MESH = plsc.VectorSubcoreMesh(
    core_axis_name="c", subcore_axis_name="s", num_cores=2, num_subcores=16)


def _hist_level(shift, masked):
    def body(bits_hbm, pvec_hbm, hist_hbm, buf, hist, pv):
        wid = lax.axis_index("s") * 2 + lax.axis_index("c")
        zeros16 = jnp.zeros((16,), jnp.int32)
        ones16 = jnp.ones((16,), jnp.int32)
        lane = lax.iota(jnp.int32, 16)

        def zb(i, carry):
            for j in range(8):
                hist[pl.ds(i * 128 + j * 16, 16)] = zeros16
            return carry

        lax.fori_loop(0, HL // 128, zb, 0)

        if masked:
            pltpu.sync_copy(pvec_hbm, pv)
            pvx = pv[...]
        base = wid * PER_W

        def cb(ci, carry):
            pltpu.sync_copy(bits_hbm.at[pl.ds(base + ci * CHUNK, CHUNK)], buf)

            def vb(vi, c2):
                for j in range(4):
                    u = buf[pl.ds(vi * 64 + j * 16, 16)]
                    b = jnp.bitwise_and(jnp.right_shift(u, shift), NB - 1)
                    idx = b * 16 + lane
                    if masked:
                        m = jnp.right_shift(u, shift + 10) == pvx
                        plsc.addupdate_scatter(hist, [idx], ones16, mask=m)
                    else:
                        plsc.addupdate_scatter(hist, [idx], ones16)
                return c2

            lax.fori_loop(0, CHUNK // 64, vb, 0)
            return carry

        lax.fori_loop(0, NCHUNK, cb, 0)
        pltpu.sync_copy(hist, hist_hbm.at[wid])

    scratch = [
        pltpu.VMEM((CHUNK,), jnp.int32),
        pltpu.VMEM((HL,), jnp.int32),
        pltpu.VMEM((16,), jnp.int32),
    ]
    return pl.kernel(
        body,
        out_type=jax.ShapeDtypeStruct((NW, HL), jnp.int32),
        mesh=_SC_MESH,
        scratch_types=scratch,
        compiler_params=pltpu.CompilerParams(needs_layout_passes=False),
    )


_hist_l0 = _hist_level(20, False)
_hist_l1 = _hist_level(10, True)
_hist_l2 = _hist_level(0, True)


def _triangulars(n):
    r = lax.broadcasted_iota(jnp.int32, (n, n), 0)
    c = lax.broadcasted_iota(jnp.int32, (n, n), 1)
    upper = (r < c).astype(jnp.float32)
    lower = (c < r).astype(jnp.float32)
    return upper, lower


def _select_body(first, h_ref, l0_ref, st_ref, out_ref, pvec_ref):
    h = jnp.sum(h_ref[...].astype(jnp.float32), axis=0)
    up, lo = _triangulars(128)
    inrow = jnp.dot(h, up, preferred_element_type=jnp.float32,
                    precision=lax.Precision.HIGHEST)
    rowsum = jnp.sum(h, axis=1, keepdims=True)
    rowoffs = jnp.dot(lo, rowsum, preferred_element_type=jnp.float32,
                      precision=lax.Precision.HIGHEST)
    cum_excl = rowoffs + inrow
    cum_incl = cum_excl + h

    if first:
        kf = jnp.clip(jnp.round(jnp.float32(N_IN) - l0_ref[0]), 0.0,
                      jnp.float32(N_IN))
        k_rem = kf.astype(jnp.int32)
    else:
        k_rem = st_ref[0]
    prefix = st_ref[1]

    r = lax.broadcasted_iota(jnp.int32, (128, 128), 0)
    c = lax.broadcasted_iota(jnp.int32, (128, 128), 1)
    fpos = r * 128 + c
    lane_end = jnp.bitwise_and(fpos, 15) == 15
    sel = lane_end & (cum_incl >= k_rem.astype(jnp.float32))
    bkt = jnp.min(jnp.where(sel, jnp.right_shift(fpos, 4),
                            jnp.int32(1 << 30)))
    clb = jnp.sum(jnp.where(fpos == bkt * 16, cum_excl, 0.0))

    out_ref[0] = k_rem - clb.astype(jnp.int32)
    p_out = prefix * NB + bkt
    out_ref[1] = p_out
    for i in range(16):
        pvec_ref[i] = p_out


def _select(first):
    return pl.pallas_call(
        functools.partial(_select_body, first),
        in_specs=[
            pl.BlockSpec(memory_space=pltpu.VMEM),
            pl.BlockSpec(memory_space=pltpu.SMEM),
            pl.BlockSpec(memory_space=pltpu.SMEM),
        ],
        out_specs=[
            pl.BlockSpec(memory_space=pltpu.SMEM),
            pl.BlockSpec(memory_space=pltpu.SMEM),
        ],
        out_shape=[
            jax.ShapeDtypeStruct((8,), jnp.int32),
            jax.ShapeDtypeStruct((16,), jnp.int32),
        ],
    )


def _k7_body(bits_ref, st_ref, out_ref, carry_ref):
    pid = pl.program_id(0)

    @pl.when(pid == 0)
    def _():
        carry_ref[0] = 0.0

    rr = st_ref[0]
    thr = st_ref[1]
    u = bits_ref[...]
    s = lax.bitcast_convert_type(u, jnp.float32)
    eq = u == thr
    eqf = eq.astype(jnp.float32)
    up128, _ = _triangulars(128)
    _, lo512 = _triangulars(BLK0)
    inrow = jnp.dot(eqf, up128, preferred_element_type=jnp.float32,
                    precision=lax.Precision.HIGHEST)
    rowsum = jnp.sum(eqf, axis=1, keepdims=True)
    rowoffs = jnp.dot(lo512, rowsum, preferred_element_type=jnp.float32,
                      precision=lax.Precision.HIGHEST)
    pre = carry_ref[0] + rowoffs + inrow
    zero = (u < thr) | (eq & (pre < rr.astype(jnp.float32)))
    out_ref[...] = jnp.where(zero, 0.0, s)
    carry_ref[0] += jnp.sum(eqf)


def _k7(bits2d, state):
    return pl.pallas_call(
        _k7_body,
        grid=(GRID0,),
        in_specs=[
            pl.BlockSpec((BLK0, 128), lambda i: (i, 0)),
            pl.BlockSpec(memory_space=pltpu.SMEM),
        ],
        out_specs=pl.BlockSpec((BLK0, 128), lambda i: (i, 0)),
        out_shape=jax.ShapeDtypeStruct((NROWS, 128), jnp.float32),
        scratch_shapes=[pltpu.SMEM((1,), jnp.float32)],
    )(bits2d, state)


@jax.jit
def kernel(log_alpha):
    xpad = jnp.pad(log_alpha, (0, NPAD - N_IN), constant_values=200.0)
    x2d = xpad.reshape(NROWS, 128)
    bits2d, l0 = _k0(x2d)
    bits1d = bits2d.reshape(NPAD)
    zvec = jnp.zeros((16,), jnp.int32)

    h0 = _hist_l0(bits1d, zvec)
    st, pv = _select(True)(h0.reshape(NW, 128, 128), l0, jnp.zeros((8,), jnp.int32))
    h1 = _hist_l1(bits1d, pv)
    st, pv = _select(False)(h1.reshape(NW, 128, 128), l0, st)
    h2 = _hist_l2(bits1d, pv)
    st, pv = _select(False)(h2.reshape(NW, 128, 128), l0, st)

    out2d = _k7(bits2d, st)
    return out2d.reshape(NPAD)[:N_IN]


# speedup vs baseline: 1.1750x; 1.1750x over previous
import functools
import math

import jax
import jax.numpy as jnp
from jax import lax
from jax.experimental import pallas as pl
from jax.experimental.pallas import tpu as pltpu
import jax.experimental.pallas.tpu_sc as plsc

N_IN = 1000000
NPAD = 1 << 20
NROWS = NPAD // 128
BETA = 2.0 / 3.0
BIAS = -BETA * math.log(0.1 / 1.1)

NW = 32
PER_W = NPAD // NW
CHUNK = 32768
NCHUNK = PER_W // CHUNK
NB = 1024
HL = NB * 16

BLK0 = 512
GRID0 = NROWS // BLK0


def _sigmoid(z):
    return 1.0 / (1.0 + jnp.exp(-z))


def _k0_body(x_ref, bits_ref, l0_ref, acc_ref):
    pid = pl.program_id(0)

    @pl.when(pid == 0)
    def _():
        acc_ref[0] = 0.0

    x = x_ref[...]
    s1 = _sigmoid(x + BIAS)
    r = lax.broadcasted_iota(jnp.int32, (BLK0, 128), 0)
    c = lax.broadcasted_iota(jnp.int32, (BLK0, 128), 1)
    flat = (pid * BLK0 + r) * 128 + c
    s1 = jnp.where(flat < N_IN, s1, 0.0)
    acc_ref[0] += jnp.sum(s1)
    bits_ref[...] = lax.bitcast_convert_type(_sigmoid((x / BETA) * 0.8),
                                             jnp.int32)

    @pl.when(pid == pl.num_programs(0) - 1)
    def _():
        l0_ref[0] = acc_ref[0]


def _k0(x2d):
    return pl.pallas_call(
        _k0_body,
        grid=(GRID0,),
        in_specs=[pl.BlockSpec((BLK0, 128), lambda i: (i, 0))],
        out_specs=[
            pl.BlockSpec((BLK0, 128), lambda i: (i, 0)),
            pl.BlockSpec(memory_space=pltpu.SMEM),
        ],
        out_shape=[
            jax.ShapeDtypeStruct((NROWS, 128), jnp.int32),
            jax.ShapeDtypeStruct((1,), jnp.float32),
        ],
        scratch_shapes=[pltpu.SMEM((1,), jnp.float32)],
    )(x2d)


_SC_MESH = plsc.VectorSubcoreMesh(
    core_axis_name="c", subcore_axis_name="s", num_cores=2, num_subcores=16)


def _hist_level(shift, masked):
    def body(bits_hbm, pvec_hbm, hist_hbm, buf, hist, pv):
        wid = lax.axis_index("s") * 2 + lax.axis_index("c")
        zeros16 = jnp.zeros((16,), jnp.int32)
        ones16 = jnp.ones((16,), jnp.int32)
        lane = lax.iota(jnp.int32, 16)

        def zb(i, carry):
            for j in range(8):
                hist[pl.ds(i * 128 + j * 16, 16)] = zeros16
            return carry

        lax.fori_loop(0, HL // 128, zb, 0)

        if masked:
            pltpu.sync_copy(pvec_hbm, pv)
            pvx = pv[...]
        base = wid * PER_W

        def cb(ci, carry):
            pltpu.sync_copy(bits_hbm.at[pl.ds(base + ci * CHUNK, CHUNK)], buf)

            def vb(vi, c2):
                for j in range(4):
                    u = buf[pl.ds(vi * 64 + j * 16, 16)]
                    b = jnp.bitwise_and(jnp.right_shift(u, shift), NB - 1)
                    idx = b * 16 + lane
                    if masked:
                        m = jnp.right_shift(u, shift + 10) == pvx
                        plsc.addupdate_scatter(hist, [idx], ones16, mask=m)
                    else:
                        plsc.addupdate_scatter(hist, [idx], ones16)
                return c2

            lax.fori_loop(0, CHUNK // 64, vb, 0)
            return carry

        lax.fori_loop(0, NCHUNK, cb, 0)
        pltpu.sync_copy(hist, hist_hbm.at[wid])

    scratch = [
        pltpu.VMEM((CHUNK,), jnp.int32),
        pltpu.VMEM((HL,), jnp.int32),
        pltpu.VMEM((16,), jnp.int32),
    ]
    return pl.kernel(
        body,
        out_type=jax.ShapeDtypeStruct((NW, HL), jnp.int32),
        mesh=_SC_MESH,
        scratch_types=scratch,
        compiler_params=pltpu.CompilerParams(needs_layout_passes=False),
    )


_hist_l0 = _hist_level(20, False)
_hist_l1 = _hist_level(10, True)
_hist_l2 = _hist_level(0, True)


def _triangulars(n):
    r = lax.broadcasted_iota(jnp.int32, (n, n), 0)
    c = lax.broadcasted_iota(jnp.int32, (n, n), 1)
    upper = (r < c).astype(jnp.float32)
    lower = (c < r).astype(jnp.float32)
    return upper, lower


def _select_body(first, h_ref, l0_ref, st_ref, out_ref, pvec_ref):
    h = jnp.sum(h_ref[...].astype(jnp.float32), axis=0)
    up, lo = _triangulars(128)
    inrow = jnp.dot(h, up, preferred_element_type=jnp.float32,
                    precision=lax.Precision.HIGHEST)
    rowsum = jnp.sum(h, axis=1, keepdims=True)
    rowoffs = jnp.dot(lo, rowsum, preferred_element_type=jnp.float32,
                      precision=lax.Precision.HIGHEST)
    cum_excl = rowoffs + inrow
    cum_incl = cum_excl + h

    if first:
        kf = jnp.clip(jnp.round(jnp.float32(N_IN) - l0_ref[0]), 0.0,
                      jnp.float32(N_IN))
        k_rem = kf.astype(jnp.int32)
    else:
        k_rem = st_ref[0]
    prefix = st_ref[1]

    r = lax.broadcasted_iota(jnp.int32, (128, 128), 0)
    c = lax.broadcasted_iota(jnp.int32, (128, 128), 1)
    fpos = r * 128 + c
    lane_end = jnp.bitwise_and(fpos, 15) == 15
    sel = lane_end & (cum_incl >= k_rem.astype(jnp.float32))
    bkt = jnp.min(jnp.where(sel, jnp.right_shift(fpos, 4),
                            jnp.int32(1 << 30)))
    clb = jnp.sum(jnp.where(fpos == bkt * 16, cum_excl, 0.0))

    out_ref[0] = k_rem - clb.astype(jnp.int32)
    p_out = prefix * NB + bkt
    out_ref[1] = p_out
    for i in range(16):
        pvec_ref[i] = p_out


def _select(first):
    return pl.pallas_call(
        functools.partial(_select_body, first),
        in_specs=[
            pl.BlockSpec(memory_space=pltpu.VMEM),
            pl.BlockSpec(memory_space=pltpu.SMEM),
            pl.BlockSpec(memory_space=pltpu.SMEM),
        ],
        out_specs=[
            pl.BlockSpec(memory_space=pltpu.SMEM),
            pl.BlockSpec(memory_space=pltpu.SMEM),
        ],
        out_shape=[
            jax.ShapeDtypeStruct((8,), jnp.int32),
            jax.ShapeDtypeStruct((16,), jnp.int32),
        ],
    )


def _k7_body(bits_ref, st_ref, out_ref, carry_ref):
    pid = pl.program_id(0)

    @pl.when(pid == 0)
    def _():
        carry_ref[0] = 0.0

    rr = st_ref[0]
    thr = st_ref[1]
    u = bits_ref[...]
    s = lax.bitcast_convert_type(u, jnp.float32)
    eq = u == thr
    eqf = eq.astype(jnp.float32)
    up128, _ = _triangulars(128)
    _, lo512 = _triangulars(BLK0)
    inrow = jnp.dot(eqf, up128, preferred_element_type=jnp.float32,
                    precision=lax.Precision.HIGHEST)
    rowsum = jnp.sum(eqf, axis=1, keepdims=True)
    rowoffs = jnp.dot(lo512, rowsum, preferred_element_type=jnp.float32,
                      precision=lax.Precision.HIGHEST)
    pre = carry_ref[0] + rowoffs + inrow
    zero = (u < thr) | (eq & (pre < rr.astype(jnp.float32)))
    out_ref[...] = jnp.where(zero, 0.0, s)
    carry_ref[0] += jnp.sum(eqf)


def _k7(bits2d, state):
    return pl.pallas_call(
        _k7_body,
        grid=(GRID0,),
        in_specs=[
            pl.BlockSpec((BLK0, 128), lambda i: (i, 0)),
            pl.BlockSpec(memory_space=pltpu.SMEM),
        ],
        out_specs=pl.BlockSpec((BLK0, 128), lambda i: (i, 0)),
        out_shape=jax.ShapeDtypeStruct((NROWS, 128), jnp.float32),
        scratch_shapes=[pltpu.SMEM((1,), jnp.float32)],
    )(bits2d, state)


@jax.jit
def kernel(log_alpha):
    xpad = jnp.pad(log_alpha, (0, NPAD - N_IN), constant_values=200.0)
    x2d = xpad.reshape(NROWS, 128)
    bits2d, l0 = _k0(x2d)
    bits1d = bits2d.reshape(NPAD)
    zvec = jnp.zeros((16,), jnp.int32)

    h0 = _hist_l0(bits1d, zvec)
    st, pv = _select(True)(h0.reshape(NW, 128, 128), l0, jnp.zeros((8,), jnp.int32))
    h1 = _hist_l1(bits1d, pv)
    st, pv = _select(False)(h1.reshape(NW, 128, 128), l0, st)
    h2 = _hist_l2(bits1d, pv)
    st, pv = _select(False)(h2.reshape(NW, 128, 128), l0, st)

    out2d = _k7(bits2d, st)
    return out2d.reshape(NPAD)[:N_IN]
